# Initial kernel scaffold; baseline (speedup 1.0000x reference)
import jax
import jax.numpy as jnp
from jax.experimental import pallas as pl


def kernel(x):
    raise NotImplementedError("write your pallas kernel here")


# speedup vs baseline: 134.1544x; 134.1544x over previous
import jax
import jax.numpy as jnp
from jax.experimental import pallas as pl
from jax.experimental.pallas import tpu as pltpu

ALPHA_ = 0.5
HB = 16


def _midmax_body(x_ref, o_ref):
    blk = x_ref[0]
    r0 = blk[:, 0]
    r1 = blk[:, 1]
    a = r0[:, :, :64]
    b = r0[:, :, 64:]
    c = r1[:, :, :64]
    d = r1[:, :, 64:]
    m1 = jnp.minimum(a, b)
    mx1 = jnp.maximum(a, b)
    m2 = jnp.minimum(c, d)
    mx2 = jnp.maximum(c, d)
    max4 = jnp.maximum(mx1, mx2)
    sec = jnp.minimum(jnp.maximum(m1, m2), jnp.minimum(mx1, mx2))
    o_ref[0] = ALPHA_ * max4 + (1.0 - ALPHA_) * jnp.maximum(sec, 0.0)


def kernel(x):
    B, H, W, C = x.shape
    Ho, Wo = H // 2, W // 2
    xr = x.reshape(B, Ho, 2, (W * C) // 128, 128)
    grid = (B, Ho // HB)
    out = pl.pallas_call(
        _midmax_body,
        grid=grid,
        in_specs=[pl.BlockSpec((1, HB, 2, (W * C) // 128, 128),
                               lambda b, h: (b, h, 0, 0, 0))],
        out_specs=pl.BlockSpec((1, HB, Wo, C), lambda b, h: (b, h, 0, 0)),
        out_shape=jax.ShapeDtypeStruct((B, Ho, Wo, C), x.dtype),
        compiler_params=pltpu.CompilerParams(
            dimension_semantics=("parallel", "arbitrary")),
    )(xr)
    return out


# speedup vs baseline: 141.2307x; 1.0527x over previous
import jax
import jax.numpy as jnp
from jax.experimental import pallas as pl
from jax.experimental.pallas import tpu as pltpu

ALPHA_ = 0.5
HB = 16


def _midmax_body(x_ref, o_ref):
    blk = x_ref[0]
    h0 = blk[:, 0]
    h1 = blk[:, 1]
    vmin = jnp.minimum(h0, h1)
    vmax = jnp.maximum(h0, h1)
    vmin4 = vmin.reshape(HB, 128, 2, 64)
    vmax4 = vmax.reshape(HB, 128, 2, 64)
    m1 = vmin4[:, :, 0, :]
    m2 = vmin4[:, :, 1, :]
    M1 = vmax4[:, :, 0, :]
    M2 = vmax4[:, :, 1, :]
    max4 = jnp.maximum(M1, M2)
    sec = jnp.minimum(jnp.maximum(m1, m2), jnp.minimum(M1, M2))
    o_ref[0] = ALPHA_ * max4 + (1.0 - ALPHA_) * jnp.maximum(sec, 0.0)


def kernel(x):
    B, H, W, C = x.shape
    Ho, Wo = H // 2, W // 2
    xr = x.reshape(B, Ho, 2, W, C)
    grid = (B, Ho // HB)
    out = pl.pallas_call(
        _midmax_body,
        grid=grid,
        in_specs=[pl.BlockSpec((1, HB, 2, W, C),
                               lambda b, h: (b, h, 0, 0, 0))],
        out_specs=pl.BlockSpec((1, HB, Wo, C), lambda b, h: (b, h, 0, 0)),
        out_shape=jax.ShapeDtypeStruct((B, Ho, Wo, C), x.dtype),
        compiler_params=pltpu.CompilerParams(
            dimension_semantics=("parallel", "arbitrary")),
    )(xr)
    return out
